# Initial kernel scaffold; baseline (speedup 1.0000x reference)
import jax
import jax.numpy as jnp
from jax.experimental import pallas as pl


def kernel(inputs, edges, W1, b1, W2, b2, ln2_g, ln2_b, Wih, Whh, bih, bhh, ln1_g, ln1_b, hbW, hbb, cbW, cbb):
    raise NotImplementedError("write your pallas kernel here")


# speedup vs baseline: 5.6047x; 5.6047x over previous
import functools

import jax
import jax.numpy as jnp
from jax import lax
from jax.experimental import pallas as pl
from jax.experimental.pallas import tpu as pltpu
from jax.experimental.pallas import tpu_sc as plsc

N = 10000
E = 320000
D = 128
T = 12
H = 128

NPAD = 10240
NB_NODE = 512
NGRID = 20
IDXW = 128
NTILE = 16
NCORE = 2
NW = NCORE * NTILE
HR = NPAD // IDXW


def _make_sc_scatter(n_slots, nb, acc_rows, drain_rows, src_slab, dst_slab,
                     gather=True):
  stripe = drain_rows // NTILE
  zpt = acc_rows // NTILE // 16
  mesh = plsc.VectorSubcoreMesh(core_axis_name="c", subcore_axis_name="s")
  scratch = [
      pltpu.VMEM((nb, IDXW), jnp.int32),
      pltpu.VMEM((nb, IDXW), jnp.int32),
      pltpu.VMEM((IDXW, H), jnp.float32),
      pltpu.VMEM((16, H), jnp.float32),
      pltpu.VMEM_SHARED((acc_rows, H), jnp.float32),
      pltpu.SemaphoreType.DMA,
  ]

  def body(table, src_all, dst_all, out, src_v, dst_v, buf, zbuf, acc, sem):
    c = lax.axis_index("c")
    s = lax.axis_index("s")

    def zrow(j, carry):
      for kk in range(H // 16):
        zbuf[j, pl.ds(16 * kk, 16)] = jnp.zeros((16,), jnp.float32)
      return carry

    lax.fori_loop(0, 16, zrow, 0)
    if not gather:
      def orow(j, carry):
        for kk in range(H // 16):
          buf[j, pl.ds(16 * kk, 16)] = jnp.ones((16,), jnp.float32)
        return carry

      lax.fori_loop(0, IDXW, orow, 0)

    for k in range(n_slots // NCORE):
      g = NCORE * k + c
      plsc.subcore_barrier()

      def zacc(j, carry):
        pltpu.sync_copy(zbuf, acc.at[pl.ds(s * (zpt * 16) + j * 16, 16)])
        return carry

      lax.fori_loop(0, zpt, zacc, 0)
      plsc.subcore_barrier()

      if gather:
        pltpu.sync_copy(src_all.at[src_slab(g, s)], src_v)
      pltpu.sync_copy(dst_all.at[dst_slab(g, s)], dst_v)

      def batch(j, carry):
        if gather:
          pltpu.async_copy(table.at[src_v.at[j]], buf, sem).wait()
        pltpu.sync_copy(buf, acc.at[dst_v.at[j]], add=True)
        return carry

      lax.fori_loop(0, nb, batch, 0)
      plsc.subcore_barrier()
      pltpu.sync_copy(acc.at[pl.ds(s * stripe, stripe)],
                      out.at[pl.ds(g * drain_rows + s * stripe, stripe)])

  out_type = jax.ShapeDtypeStruct((n_slots * drain_rows, H), jnp.float32)
  if gather:
    k = functools.partial(pl.kernel, mesh=mesh, out_type=out_type,
                          scratch_types=scratch)(body)
    return k
  def body_ng(dst_all, out, src_v, dst_v, buf, zbuf, acc, sem):
    return body(None, None, dst_all, out, src_v, dst_v, buf, zbuf, acc, sem)

  return functools.partial(pl.kernel, mesh=mesh, out_type=out_type,
                           scratch_types=scratch)(body_ng)


def _dinv_body(g_ref, o_ref):
  o_ref[...] = lax.rsqrt(1.0 + g_ref[0][:, 0:1])


def _k1_body(x_ref, w_ref, d_ref, o_ref):
  w = w_ref[...]
  d = d_ref[...]
  for t in range(T):
    o_ref[t] = jnp.dot(x_ref[:, t, :], w, preferred_element_type=jnp.float32) * d


def _k2_body(g_ref, p_ref, d_ref, b_ref, w_ref, o_ref):
  c1 = d_ref[...] * (g_ref[0] + p_ref[0]) + b_ref[...]
  h = jnp.maximum(c1, 0.0)
  o_ref[0] = jnp.dot(h, w_ref[...], preferred_element_type=jnp.float32) * d_ref[...]


def _ln(x, g, b):
  m = jnp.mean(x, axis=-1, keepdims=True)
  v = jnp.mean(x * x, axis=-1, keepdims=True) - m * m
  return (x - m) * lax.rsqrt(v + 1e-5) * g + b


def _k3_body(g2_ref, p2_ref, d_ref, b2_ref, ln2g_ref, ln2b_ref, wih_ref,
             whh_ref, bsum_ref, ln1g_ref, ln1b_ref, hbw_ref, hbb_ref,
             cbw_ref, cbb_ref, series_ref, hidden_ref, cell_ref):
  d = d_ref[...]
  x = d * (g2_ref[...] + p2_ref[...]) + b2_ref[...]
  xn = _ln(x, ln2g_ref[...], ln2b_ref[...])
  wih = wih_ref[...]
  whh = whh_ref[...]
  bsum = bsum_ref[...]
  ln1g = ln1g_ref[...]
  ln1b = ln1b_ref[...]
  h = jnp.zeros((NB_NODE, H), jnp.float32)
  c = jnp.zeros((NB_NODE, H), jnp.float32)
  for t in range(T):
    gates = (jnp.dot(xn[t], wih, preferred_element_type=jnp.float32)
             + jnp.dot(h, whh, preferred_element_type=jnp.float32) + bsum)
    ig = jax.nn.sigmoid(gates[:, 0:H])
    fg = jax.nn.sigmoid(gates[:, H:2 * H])
    gg = jnp.tanh(gates[:, 2 * H:3 * H])
    og = jax.nn.sigmoid(gates[:, 3 * H:4 * H])
    c = fg * c + ig * gg
    h = og * jnp.tanh(c)
    series_ref[:, t, :] = _ln(h, ln1g, ln1b)
  hidden_ref[0] = jnp.tanh(
      jnp.dot(h, hbw_ref[...], preferred_element_type=jnp.float32) + hbb_ref[...])
  cell_ref[0] = jnp.dot(c, cbw_ref[...], preferred_element_type=jnp.float32) + cbb_ref[...]


def kernel(inputs, edges, W1, b1, W2, b2, ln2_g, ln2_b, Wih, Whh, bih, bhh,
           ln1_g, ln1_b, hbW, hbb, cbW, cbb):
  f32 = jnp.float32
  src = edges[0]
  dst = edges[1]

  nb_c = (E // NTILE + IDXW - 1) // IDXW
  per_tile_c = nb_c * IDXW
  pad_c = NTILE * per_tile_c - E
  padv = jnp.arange(pad_c, dtype=jnp.int32)
  src_c = jnp.concatenate([src, padv % N]).reshape(NTILE, per_tile_c)
  dst_c = jnp.concatenate([dst, N + (padv % (NPAD - N))]).reshape(
      NTILE, per_tile_c)
  toff = (jnp.arange(T, dtype=jnp.int32) * N)[:, None, None]
  src_conv = (src_c[None] + toff).reshape(T * NTILE, nb_c, IDXW)
  HALF = NPAD // 2
  disc_c = HALF + (jnp.arange(per_tile_c, dtype=jnp.int32) % 512)[None]
  dst_conv = jnp.stack([
      jnp.where((dst_c >= HALF * h) & (dst_c < HALF * (h + 1)),
                dst_c - HALF * h, disc_c) for h in range(2)]).reshape(
                    2 * NTILE, nb_c, IDXW)

  CH = 2048
  disc_d = CH + (jnp.arange(per_tile_c, dtype=jnp.int32) % 256)[None]
  dst_deg = jnp.stack([
      jnp.where((dst_c >= CH * q) & (dst_c < CH * (q + 1)), dst_c - CH * q,
                disc_d) for q in range(6)]).reshape(6 * NTILE, nb_c, IDXW)

  deg6 = _make_sc_scatter(
      6, nb_c, 2304, CH,
      None, lambda g, s: g * NTILE + s, gather=False)(dst_deg).reshape(
          6, CH, H)

  dinv = pl.pallas_call(
      _dinv_body,
      grid=(NGRID,),
      in_specs=[pl.BlockSpec((1, NB_NODE, H), lambda i: (i // 4, i % 4, 0))],
      out_specs=pl.BlockSpec((NB_NODE, 1), lambda i: (i, 0)),
      out_shape=jax.ShapeDtypeStruct((NPAD, 1), f32),
  )(deg6)

  p1 = pl.pallas_call(
      _k1_body,
      grid=(NGRID,),
      in_specs=[
          pl.BlockSpec((NB_NODE, T, D), lambda i: (i, 0, 0)),
          pl.BlockSpec((D, H), lambda i: (0, 0)),
          pl.BlockSpec((NB_NODE, 1), lambda i: (i, 0)),
      ],
      out_specs=pl.BlockSpec((T, NB_NODE, H), lambda i: (0, i, 0)),
      out_shape=jax.ShapeDtypeStruct((T, N, H), f32),
  )(inputs, W1, dinv)

  conv = _make_sc_scatter(2 * T, nb_c, 5632, HALF,
                          lambda g, s: (g // 2) * NTILE + s,
                          lambda g, s: (g % 2) * NTILE + s)

  g1 = conv(p1.reshape(T * N, H), src_conv, dst_conv).reshape(T, NPAD, H)

  p2 = pl.pallas_call(
      _k2_body,
      grid=(T, NGRID),
      in_specs=[
          pl.BlockSpec((1, NB_NODE, H), lambda t, i: (t, i, 0)),
          pl.BlockSpec((1, NB_NODE, H), lambda t, i: (t, i, 0)),
          pl.BlockSpec((NB_NODE, 1), lambda t, i: (i, 0)),
          pl.BlockSpec((1, H), lambda t, i: (0, 0)),
          pl.BlockSpec((H, H), lambda t, i: (0, 0)),
      ],
      out_specs=pl.BlockSpec((1, NB_NODE, H), lambda t, i: (t, i, 0)),
      out_shape=jax.ShapeDtypeStruct((T, N, H), f32),
  )(g1, p1, dinv, b1.reshape(1, H), W2)

  g2 = conv(p2.reshape(T * N, H), src_conv, dst_conv).reshape(T, NPAD, H)

  series, hidden, cell = pl.pallas_call(
      _k3_body,
      grid=(NGRID,),
      in_specs=[
          pl.BlockSpec((T, NB_NODE, H), lambda i: (0, i, 0)),
          pl.BlockSpec((T, NB_NODE, H), lambda i: (0, i, 0)),
          pl.BlockSpec((NB_NODE, 1), lambda i: (i, 0)),
          pl.BlockSpec((1, H), lambda i: (0, 0)),
          pl.BlockSpec((1, H), lambda i: (0, 0)),
          pl.BlockSpec((1, H), lambda i: (0, 0)),
          pl.BlockSpec((H, 4 * H), lambda i: (0, 0)),
          pl.BlockSpec((H, 4 * H), lambda i: (0, 0)),
          pl.BlockSpec((1, 4 * H), lambda i: (0, 0)),
          pl.BlockSpec((1, H), lambda i: (0, 0)),
          pl.BlockSpec((1, H), lambda i: (0, 0)),
          pl.BlockSpec((H, H), lambda i: (0, 0)),
          pl.BlockSpec((1, H), lambda i: (0, 0)),
          pl.BlockSpec((H, H), lambda i: (0, 0)),
          pl.BlockSpec((1, H), lambda i: (0, 0)),
      ],
      out_specs=[
          pl.BlockSpec((NB_NODE, T, H), lambda i: (i, 0, 0)),
          pl.BlockSpec((1, NB_NODE, H), lambda i: (0, i, 0)),
          pl.BlockSpec((1, NB_NODE, H), lambda i: (0, i, 0)),
      ],
      out_shape=[
          jax.ShapeDtypeStruct((N, T, H), f32),
          jax.ShapeDtypeStruct((1, N, H), f32),
          jax.ShapeDtypeStruct((1, N, H), f32),
      ],
  )(g2, p2, dinv, b2.reshape(1, H), ln2_g.reshape(1, H), ln2_b.reshape(1, H),
    Wih.T, Whh.T, (bih + bhh).reshape(1, 4 * H), ln1_g.reshape(1, H),
    ln1_b.reshape(1, H), hbW.T, hbb.reshape(1, H), cbW.T, cbb.reshape(1, H))

  return (series, hidden, cell)


# speedup vs baseline: 6.4334x; 1.1479x over previous
import functools

import jax
import jax.numpy as jnp
from jax import lax
from jax.experimental import pallas as pl
from jax.experimental.pallas import tpu as pltpu
from jax.experimental.pallas import tpu_sc as plsc

N = 10000
E = 320000
D = 128
T = 12
H = 128

NPAD = 10240
CH3 = 3584
ACC = CH3 + 512
NB_NODE = 512
NGRID = 20
IDXW = 128
NTILE = 16
NCORE = 2
NSLOT = 3 * T + 4


def _make_sc_conv(nb):
  stripe = CH3 // NTILE
  zrows = ACC // NTILE // 2
  mesh = plsc.VectorSubcoreMesh(core_axis_name="c", subcore_axis_name="s")

  @functools.partial(
      pl.kernel,
      mesh=mesh,
      out_type=jax.ShapeDtypeStruct((NSLOT * CH3, H), jnp.float32),
      scratch_types=[
          pltpu.VMEM((16,), jnp.int32),
          pltpu.VMEM((nb, IDXW), jnp.int32),
          pltpu.VMEM((nb, IDXW), jnp.int32),
          pltpu.VMEM((2 * IDXW, H), jnp.float32),
          pltpu.VMEM((zrows, H), jnp.float32),
          pltpu.VMEM_SHARED((ACC, H), jnp.float32),
          pltpu.SemaphoreType.DMA,
      ],
  )
  def sc_conv(mode_hbm, table, src_all, dst_all, out, mode_v, src_v, dst_v,
              buf, zbuf, acc, sema):
    c = lax.axis_index("c")
    s = lax.axis_index("s")
    pltpu.sync_copy(mode_hbm, mode_v)
    conv_on = mode_v[pl.ds(0, 16)][0] == 1

    def zrow(j, carry):
      for kk in range(H // 16):
        zbuf[j, pl.ds(16 * kk, 16)] = jnp.zeros((16,), jnp.float32)
      return carry

    lax.fori_loop(0, zrows, zrow, 0)

    @pl.when(jnp.logical_not(conv_on))
    def _():
      def orow(j, carry):
        for kk in range(H // 16):
          buf[j, pl.ds(16 * kk, 16)] = jnp.ones((16,), jnp.float32)
        return carry

      lax.fori_loop(0, IDXW, orow, 0)

    def slot(k, carry):
      is_deg = k >= 3 * T // 2
      g = 2 * k + c
      t_idx = g // 3
      d_idx = jnp.where(is_deg, jnp.minimum(g - 3 * T, 2), g % 3)
      active = jnp.logical_xor(conv_on, is_deg)
      plsc.subcore_barrier()

      @pl.when(active)
      def _():
        pltpu.sync_copy(zbuf, acc.at[pl.ds(s * (zrows * 2), zrows)])
        pltpu.sync_copy(zbuf, acc.at[pl.ds(s * (zrows * 2) + zrows, zrows)])

      plsc.subcore_barrier()

      @pl.when(active)
      def _():
        pltpu.sync_copy(dst_all.at[d_idx * NTILE + s], dst_v)

      @pl.when(active & jnp.logical_not(is_deg))
      def _():
        pltpu.sync_copy(src_all.at[t_idx * NTILE + s], src_v)
        pltpu.async_copy(table.at[src_v.at[0]], buf.at[pl.ds(0, IDXW)], sema)

        def batch(j, carry):
          jn = jnp.minimum(j + 1, nb - 1)
          offn = ((j + 1) % 2) * IDXW
          off = (j % 2) * IDXW
          pltpu.async_copy(table.at[src_v.at[jn]],
                           buf.at[pl.ds(offn, IDXW)], sema)
          pltpu.make_async_copy(table.at[src_v.at[j]],
                                buf.at[pl.ds(off, IDXW)], sema).wait()
          pltpu.sync_copy(buf.at[pl.ds(off, IDXW)], acc.at[dst_v.at[j]],
                          add=True)
          return carry

        lax.fori_loop(0, nb, batch, 0)
        pltpu.make_async_copy(table.at[src_v.at[nb - 1]],
                              buf.at[pl.ds(0, IDXW)], sema).wait()

      @pl.when(active & is_deg)
      def _():
        def batchd(j, carry):
          pltpu.sync_copy(buf.at[pl.ds(0, IDXW)], acc.at[dst_v.at[j]],
                          add=True)
          return carry

        lax.fori_loop(0, nb, batchd, 0)

      plsc.subcore_barrier()

      @pl.when(active)
      def _():
        pltpu.sync_copy(acc.at[pl.ds(s * stripe, stripe)],
                        out.at[pl.ds(g * CH3 + s * stripe, stripe)])

      return carry

    lax.fori_loop(0, 3 * T // 2 + 2, slot, 0)

  return sc_conv


def _dinv_body(g_ref, o_ref):
  o_ref[...] = lax.rsqrt(1.0 + g_ref[0][:, 0:1])


def _k1_body(x_ref, w_ref, d_ref, o_ref):
  w = w_ref[...]
  d = d_ref[...]
  for t in range(T):
    o_ref[t] = jnp.dot(x_ref[:, t, :], w, preferred_element_type=jnp.float32) * d


def _k2_body(g_ref, p_ref, d_ref, b_ref, w_ref, o_ref):
  c1 = d_ref[...] * (g_ref[0] + p_ref[0]) + b_ref[...]
  h = jnp.maximum(c1, 0.0)
  o_ref[0] = jnp.dot(h, w_ref[...], preferred_element_type=jnp.float32) * d_ref[...]


def _ln(x, g, b):
  m = jnp.mean(x, axis=-1, keepdims=True)
  v = jnp.mean(x * x, axis=-1, keepdims=True) - m * m
  return (x - m) * lax.rsqrt(v + 1e-5) * g + b


def _k3_body(*refs):
  g2_refs = refs[0:T]
  (p2_ref, d_ref, b2_ref, ln2g_ref, ln2b_ref, wih_ref, whh_ref, bsum_ref,
   ln1g_ref, ln1b_ref, hbw_ref, hbb_ref, cbw_ref, cbb_ref, series_ref,
   hidden_ref, cell_ref) = refs[T:]
  d = d_ref[...]
  ln2g = ln2g_ref[...]
  ln2b = ln2b_ref[...]
  wih = wih_ref[...]
  whh = whh_ref[...]
  bsum = bsum_ref[...]
  ln1g = ln1g_ref[...]
  ln1b = ln1b_ref[...]
  h = jnp.zeros((NB_NODE, H), jnp.float32)
  c = jnp.zeros((NB_NODE, H), jnp.float32)
  for t in range(T):
    x = d * (g2_refs[t][0] + p2_ref[t]) + b2_ref[...]
    xn = _ln(x, ln2g, ln2b)
    gates = (jnp.dot(xn, wih, preferred_element_type=jnp.float32)
             + jnp.dot(h, whh, preferred_element_type=jnp.float32) + bsum)
    ig = jax.nn.sigmoid(gates[:, 0:H])
    fg = jax.nn.sigmoid(gates[:, H:2 * H])
    gg = jnp.tanh(gates[:, 2 * H:3 * H])
    og = jax.nn.sigmoid(gates[:, 3 * H:4 * H])
    c = fg * c + ig * gg
    h = og * jnp.tanh(c)
    series_ref[:, t, :] = _ln(h, ln1g, ln1b)
  hidden_ref[0] = jnp.tanh(
      jnp.dot(h, hbw_ref[...], preferred_element_type=jnp.float32) + hbb_ref[...])
  cell_ref[0] = jnp.dot(c, cbw_ref[...], preferred_element_type=jnp.float32) + cbb_ref[...]


def kernel(inputs, edges, W1, b1, W2, b2, ln2_g, ln2_b, Wih, Whh, bih, bhh,
           ln1_g, ln1_b, hbW, hbb, cbW, cbb):
  f32 = jnp.float32
  src = edges[0]
  dst = edges[1]

  nb_c = 2 * ((E // NTILE + 2 * IDXW - 1) // (2 * IDXW))
  per_tile_c = nb_c * IDXW
  pad_c = NTILE * per_tile_c - E
  padv = jnp.arange(pad_c, dtype=jnp.int32)
  src_c = jnp.concatenate([src, padv % N]).reshape(NTILE, per_tile_c)
  dst_c = jnp.concatenate([dst, N + (padv % (NPAD - N))]).reshape(
      NTILE, per_tile_c)
  toff = (jnp.arange(T, dtype=jnp.int32) * N)[:, None, None]
  src_conv = (src_c[None] + toff).reshape(T * NTILE, nb_c, IDXW)
  disc = CH3 + (jnp.arange(per_tile_c, dtype=jnp.int32) % 512)[None]
  dst_conv = jnp.stack([
      jnp.where((dst_c >= CH3 * q) & (dst_c < CH3 * (q + 1)),
                dst_c - CH3 * q, disc) for q in range(3)]).reshape(
                    3 * NTILE, nb_c, IDXW)

  convk = _make_sc_conv(nb_c)
  m_deg = jnp.zeros((16,), jnp.int32)
  m_conv = jnp.ones((16,), jnp.int32)

  deg_out = convk(m_deg, inputs.reshape(N * T, D), src_conv,
                  dst_conv).reshape(NSLOT, CH3, H)

  dinv = pl.pallas_call(
      _dinv_body,
      grid=(NGRID,),
      in_specs=[pl.BlockSpec((1, NB_NODE, H),
                             lambda i: (3 * T + i // 7, i % 7, 0))],
      out_specs=pl.BlockSpec((NB_NODE, 1), lambda i: (i, 0)),
      out_shape=jax.ShapeDtypeStruct((NPAD, 1), f32),
  )(deg_out)

  p1 = pl.pallas_call(
      _k1_body,
      grid=(NGRID,),
      in_specs=[
          pl.BlockSpec((NB_NODE, T, D), lambda i: (i, 0, 0)),
          pl.BlockSpec((D, H), lambda i: (0, 0)),
          pl.BlockSpec((NB_NODE, 1), lambda i: (i, 0)),
      ],
      out_specs=pl.BlockSpec((T, NB_NODE, H), lambda i: (0, i, 0)),
      out_shape=jax.ShapeDtypeStruct((T, N, H), f32),
  )(inputs, W1, dinv)

  g1 = convk(m_conv, p1.reshape(T * N, H), src_conv, dst_conv).reshape(
      NSLOT, CH3, H)

  p2 = pl.pallas_call(
      _k2_body,
      grid=(T, NGRID),
      in_specs=[
          pl.BlockSpec((1, NB_NODE, H),
                       lambda t, i: (3 * t + i // 7, i % 7, 0)),
          pl.BlockSpec((1, NB_NODE, H), lambda t, i: (t, i, 0)),
          pl.BlockSpec((NB_NODE, 1), lambda t, i: (i, 0)),
          pl.BlockSpec((1, H), lambda t, i: (0, 0)),
          pl.BlockSpec((H, H), lambda t, i: (0, 0)),
      ],
      out_specs=pl.BlockSpec((1, NB_NODE, H), lambda t, i: (t, i, 0)),
      out_shape=jax.ShapeDtypeStruct((T, N, H), f32),
  )(g1, p1, dinv, b1.reshape(1, H), W2)

  g2 = convk(m_conv, p2.reshape(T * N, H), src_conv, dst_conv).reshape(
      NSLOT, CH3, H)

  g2_specs = [
      pl.BlockSpec((1, NB_NODE, H),
                   functools.partial(lambda t, i: (3 * t + i // 7, i % 7, 0),
                                     t)) for t in range(T)
  ]
  series, hidden, cell = pl.pallas_call(
      _k3_body,
      grid=(NGRID,),
      in_specs=g2_specs + [
          pl.BlockSpec((T, NB_NODE, H), lambda i: (0, i, 0)),
          pl.BlockSpec((NB_NODE, 1), lambda i: (i, 0)),
          pl.BlockSpec((1, H), lambda i: (0, 0)),
          pl.BlockSpec((1, H), lambda i: (0, 0)),
          pl.BlockSpec((1, H), lambda i: (0, 0)),
          pl.BlockSpec((H, 4 * H), lambda i: (0, 0)),
          pl.BlockSpec((H, 4 * H), lambda i: (0, 0)),
          pl.BlockSpec((1, 4 * H), lambda i: (0, 0)),
          pl.BlockSpec((1, H), lambda i: (0, 0)),
          pl.BlockSpec((1, H), lambda i: (0, 0)),
          pl.BlockSpec((H, H), lambda i: (0, 0)),
          pl.BlockSpec((1, H), lambda i: (0, 0)),
          pl.BlockSpec((H, H), lambda i: (0, 0)),
          pl.BlockSpec((1, H), lambda i: (0, 0)),
      ],
      out_specs=[
          pl.BlockSpec((NB_NODE, T, H), lambda i: (i, 0, 0)),
          pl.BlockSpec((1, NB_NODE, H), lambda i: (0, i, 0)),
          pl.BlockSpec((1, NB_NODE, H), lambda i: (0, i, 0)),
      ],
      out_shape=[
          jax.ShapeDtypeStruct((N, T, H), f32),
          jax.ShapeDtypeStruct((1, N, H), f32),
          jax.ShapeDtypeStruct((1, N, H), f32),
      ],
  )(*([g2] * T), p2, dinv, b2.reshape(1, H),
    ln2_g.reshape(1, H), ln2_b.reshape(1, H), Wih.T, Whh.T,
    (bih + bhh).reshape(1, 4 * H), ln1_g.reshape(1, H), ln1_b.reshape(1, H),
    hbW.T, hbb.reshape(1, H), cbW.T, cbb.reshape(1, H))

  return (series, hidden, cell)
